# baseline (speedup 1.0000x reference)
import jax
import jax.numpy as jnp
from jax import lax
from jax.experimental import pallas as pl

N_NODES = 10000
E = 320000
D_OUT = 128
FAN_IN = 272

BLK_E = 2560
NB = E // BLK_E


def _mm_body(x_ref, wt_ref, wa_ref, feat_ref, attn_ref):
    x = x_ref[...]
    wt = wt_ref[...]
    feat = lax.dot_general(x, wt, (((1,), (1,)), ((), ())),
                           preferred_element_type=jnp.float32)
    feat_ref[...] = feat
    wa = wa_ref[...]
    z = lax.dot_general(wa, feat, (((1,), (1,)), ((), ())),
                        preferred_element_type=jnp.float32)
    sp = jnp.maximum(z, 0.0) + jnp.log1p(jnp.exp(-jnp.abs(z)))
    attn_ref[...] = (z * jnp.tanh(sp))[None]


def _matmul_attn(x, wt, wa):
    return pl.pallas_call(
        _mm_body,
        grid=(NB,),
        in_specs=[
            pl.BlockSpec((BLK_E, FAN_IN), lambda i: (i, 0)),
            pl.BlockSpec((D_OUT, FAN_IN), lambda i: (0, 0)),
            pl.BlockSpec((1, D_OUT), lambda i: (0, 0)),
        ],
        out_specs=[
            pl.BlockSpec((BLK_E, D_OUT), lambda i: (i, 0)),
            pl.BlockSpec((1, 1, BLK_E), lambda i: (i, 0, 0)),
        ],
        out_shape=[
            jax.ShapeDtypeStruct((E, D_OUT), jnp.float32),
            jax.ShapeDtypeStruct((NB, 1, BLK_E), jnp.float32),
        ],
    )(x, wt, wa)


@jax.jit
def _run(triple_features, indices, W_triple, W_attn):
    row = indices[0].astype(jnp.int32)
    col = indices[1].astype(jnp.int32)

    feat, attn3d = _matmul_attn(triple_features, W_triple, W_attn)
    attn = attn3d.reshape(E)

    lin = row * N_NODES + col
    order = jnp.argsort(lin).astype(jnp.int32)
    sl = jnp.take(lin, order)
    iota = jnp.arange(E, dtype=jnp.int32)
    boundary = jnp.concatenate([jnp.ones((1,), jnp.bool_), sl[1:] != sl[:-1]])
    row_s = sl // N_NODES

    def run_bounds(bnd):
        nb = jnp.concatenate([bnd[1:], jnp.ones((1,), jnp.bool_)])
        start = lax.cummax(jnp.where(bnd, iota, 0))
        end = lax.cummin(jnp.where(nb, iota, E - 1), reverse=True)
        return start, end

    def seg_sum(vals, start, end):
        c1 = jnp.concatenate([jnp.zeros((1,), jnp.float32), jnp.cumsum(vals)])
        hi = c1.at[end + 1].get(indices_are_sorted=True,
                                mode="promise_in_bounds")
        lo = c1.at[start].get(indices_are_sorted=True,
                              mode="promise_in_bounds")
        return hi - lo

    a_s = attn.at[order].get(mode="promise_in_bounds", unique_indices=True)
    cs, ce = run_bounds(boundary)
    A = seg_sum(a_s, cs, ce)
    rbnd = jnp.concatenate(
        [jnp.ones((1,), jnp.bool_), row_s[1:] != row_s[:-1]])
    rs, re_ = run_bounds(rbnd)
    rid = jnp.cumsum(rbnd.astype(jnp.int32)) - 1
    M = jax.ops.segment_max(A, rid, num_segments=N_NODES,
                            indices_are_sorted=True)
    m = M.at[rid].get(indices_are_sorted=True, mode="promise_in_bounds")
    g = jnp.exp(A - m)
    D = seg_sum(g * boundary.astype(jnp.float32), rs, re_)
    w = g / D
    feat_s = feat.at[order].get(mode="promise_in_bounds", unique_indices=True)
    out = jnp.zeros((N_NODES, D_OUT), jnp.float32).at[row_s].add(
        w[:, None] * feat_s, indices_are_sorted=True)
    return out


def kernel(triple_features, sparse_triple_adjacency_list_indices, W_triple, W_attn):
    return _run(triple_features, sparse_triple_adjacency_list_indices,
                W_triple, W_attn)


# speedup vs baseline: 1.5609x; 1.5609x over previous
import jax
import jax.numpy as jnp
from jax import lax
from jax.experimental import pallas as pl

N_NODES = 10000
E = 320000
D_OUT = 128
FAN_IN = 272

BLK_E = 2560
NB = E // BLK_E


def _mm_body(x_ref, wt_ref, wa_ref, feat_ref, attn_ref):
    x = x_ref[...]
    wt = wt_ref[...]
    feat = lax.dot_general(x, wt, (((1,), (1,)), ((), ())),
                           preferred_element_type=jnp.float32)
    feat_ref[...] = feat
    wa = wa_ref[...]
    z = lax.dot_general(wa, feat, (((1,), (1,)), ((), ())),
                        preferred_element_type=jnp.float32)
    sp = jnp.maximum(z, 0.0) + jnp.log1p(jnp.exp(-jnp.abs(z)))
    attn_ref[...] = (z * jnp.tanh(sp))[None]


def _matmul_attn(x, wt, wa):
    return pl.pallas_call(
        _mm_body,
        grid=(NB,),
        in_specs=[
            pl.BlockSpec((BLK_E, FAN_IN), lambda i: (i, 0)),
            pl.BlockSpec((D_OUT, FAN_IN), lambda i: (0, 0)),
            pl.BlockSpec((1, D_OUT), lambda i: (0, 0)),
        ],
        out_specs=[
            pl.BlockSpec((BLK_E, D_OUT), lambda i: (i, 0)),
            pl.BlockSpec((1, 1, BLK_E), lambda i: (i, 0, 0)),
        ],
        out_shape=[
            jax.ShapeDtypeStruct((E, D_OUT), jnp.float32),
            jax.ShapeDtypeStruct((NB, 1, BLK_E), jnp.float32),
        ],
    )(x, wt, wa)


@jax.jit
def _run(triple_features, indices, W_triple, W_attn):
    row = indices[0].astype(jnp.int32)
    col = indices[1].astype(jnp.int32)

    feat, attn3d = _matmul_attn(triple_features, W_triple, W_attn)
    attn = attn3d.reshape(E)

    lin = row * N_NODES + col
    order = jnp.argsort(lin).astype(jnp.int32)
    sl = jnp.take(lin, order)
    iota = jnp.arange(E, dtype=jnp.int32)
    boundary = jnp.concatenate([jnp.ones((1,), jnp.bool_), sl[1:] != sl[:-1]])
    row_s = sl // N_NODES

    def run_bounds(bnd):
        nb = jnp.concatenate([bnd[1:], jnp.ones((1,), jnp.bool_)])
        start = lax.cummax(jnp.where(bnd, iota, 0))
        end = lax.cummin(jnp.where(nb, iota, E - 1), reverse=True)
        return start, end

    def seg_sum(vals, start, end):
        c1 = jnp.concatenate([jnp.zeros((1,), jnp.float32), jnp.cumsum(vals)])
        hi = c1.at[end + 1].get(indices_are_sorted=True,
                                mode="promise_in_bounds")
        lo = c1.at[start].get(indices_are_sorted=True,
                              mode="promise_in_bounds")
        return hi - lo

    a_s = attn.at[order].get(mode="promise_in_bounds", unique_indices=True)
    cs, ce = run_bounds(boundary)
    A = seg_sum(a_s, cs, ce)
    rbnd = jnp.concatenate(
        [jnp.ones((1,), jnp.bool_), row_s[1:] != row_s[:-1]])
    rs, re_ = run_bounds(rbnd)
    rid = jnp.cumsum(rbnd.astype(jnp.int32)) - 1
    lo = jnp.min(A)
    hi = jnp.max(A)
    scale = jnp.where(hi > lo, 131072.0 / (hi - lo), 1.0)
    aq = ((A - lo) * scale).astype(jnp.int32)
    q = aq + rid * 131074
    mq = lax.cummax(q).at[re_].get(indices_are_sorted=True,
                                   mode="promise_in_bounds")
    m = lo + (mq - rid * 131074).astype(jnp.float32) / scale
    g = jnp.exp(A - m)
    D = seg_sum(g * boundary.astype(jnp.float32), rs, re_)
    w = g / D
    feat_s = feat.at[order].get(mode="promise_in_bounds", unique_indices=True)
    out = jnp.zeros((N_NODES, D_OUT), jnp.float32).at[row_s].add(
        w[:, None] * feat_s, indices_are_sorted=True)
    return out


def kernel(triple_features, sparse_triple_adjacency_list_indices, W_triple, W_attn):
    return _run(triple_features, sparse_triple_adjacency_list_indices,
                W_triple, W_attn)
